# baseline (speedup 1.0000x reference)
import functools

import jax
import jax.numpy as jnp
from jax import lax
from jax.experimental import pallas as pl
from jax.experimental.pallas import tpu as pltpu
from jax.experimental.pallas import tpu_sc as plsc

N = 10000
NP = 10240
B = 8
C = 32
E = 160000
EP = 163840
CHUNKS = EP // 16 // 128
BLKN = 1024
GRID = NP // BLKN
Q = 64


@functools.cache
def _mesh():
    return plsc.VectorSubcoreMesh(core_axis_name="c", subcore_axis_name="s")


def _sc_degree(dstp):
    def body(dst_ref, hist_ref, acc, zbuf, didx, ones, tb, isem, ssem):
        c = lax.axis_index("c")
        s = lax.axis_index("s")
        ebase = pl.multiple_of(s * (EP // 16), 128)
        hidx = pltpu.async_copy(dst_ref.at[pl.ds(ebase, EP // 16)], didx,
                                isem)
        for i in range(8):
            zbuf[pl.ds(i * 16, 16)] = jnp.zeros((16,), jnp.float32)
            ones[pl.ds(i * 16, 16)] = jnp.ones((16,), jnp.float32)
        for i in range(5):
            off = pl.multiple_of(s * 640 + i * 128, 128)
            pltpu.sync_copy(zbuf, acc.at[pl.ds(off, 128)])
        hidx.wait()
        plsc.subcore_barrier()

        hs = []
        for k in range(CHUNKS):
            off = pl.multiple_of(k * 128, 128)
            hs.append(pltpu.async_copy(ones, acc.at[didx.at[pl.ds(off, 128)]],
                                       ssem, add=True))
        for h in hs:
            h.wait()
        plsc.subcore_barrier()

        off = pl.multiple_of(c * (NP // 2) + s * 320, 64)
        pltpu.sync_copy(acc.at[pl.ds(off, 320)], tb)
        pltpu.sync_copy(tb, hist_ref.at[pl.ds(off, 320)])

    f = pl.kernel(
        body,
        out_type=jax.ShapeDtypeStruct((NP,), jnp.float32),
        mesh=_mesh(),
        scratch_types=[
            pltpu.VMEM_SHARED((NP,), jnp.float32),
            pltpu.VMEM((128,), jnp.float32),
            pltpu.VMEM((EP // 16,), jnp.int32),
            pltpu.VMEM((128,), jnp.float32),
            pltpu.VMEM((320,), jnp.float32),
            pltpu.SemaphoreType.DMA,
            pltpu.SemaphoreType.DMA,
        ],
    )
    return f(dstp)


_D = 4
_LAG = 2
_CW = 256
_NCH = EP // 16 // _CW


def _sc_segsum(u0, u1, u2, u3, srcp, dstp):
    def body(u0_ref, u1_ref, u2_ref, u3_ref, src_ref, dst_ref,
             a0_ref, a1_ref, a2_ref, a3_ref,
             acc, buf, zbuf, sidx, didx, isem,
             g0, g1, g2, g3, g4, s0, s1, s2, s3, s4):
        c = lax.axis_index("c")
        s = lax.axis_index("s")
        ebase = pl.multiple_of(s * (EP // 16), 128)
        roff = pl.multiple_of(s * 640, 64)
        gsems = (g0, g1, g2, g3, g4)
        ssems = (s0, s1, s2, s3, s4)

        hsrc = pltpu.async_copy(src_ref.at[pl.ds(ebase, EP // 16)], sidx,
                                isem)
        hdst = pltpu.async_copy(dst_ref.at[pl.ds(ebase, EP // 16)], didx,
                                isem)

        for j in range(Q // 16):
            for i in range(16):
                zbuf[i, pl.ds(j * 16, 16)] = jnp.zeros((16,), jnp.float32)
        hsrc.wait()
        hdst.wait()

        def quarter(u_ref, out_ref):
            for i in range(40):
                off = pl.multiple_of(s * 640 + i * 16, 16)
                pltpu.sync_copy(zbuf, acc.at[pl.ds(off, 16)])
            plsc.subcore_barrier()

            gh = [None] * _NCH
            sh = [None] * _NCH

            def gissue(k):
                slot = k % _D
                off = pl.multiple_of(k * _CW, 128)
                return pltpu.async_copy(
                    u_ref.at[sidx.at[pl.ds(off, _CW)]],
                    buf.at[pl.ds(slot * _CW, _CW)], gsems[slot])

            def sissue(k):
                slot = k % _D
                off = pl.multiple_of(k * _CW, 128)
                return pltpu.async_copy(
                    buf.at[pl.ds(slot * _CW, _CW)],
                    acc.at[didx.at[pl.ds(off, _CW)]], ssems[slot], add=True)

            for k in range(_NCH):
                if k >= _D:
                    sh[k - _D].wait()
                gh[k] = gissue(k)
                j = k - _LAG
                if j >= 0:
                    gh[j].wait()
                    sh[j] = sissue(j)
            for j in range(_NCH - _LAG, _NCH):
                gh[j].wait()
                sh[j] = sissue(j)
            for j in range(_NCH - _D, _NCH):
                sh[j].wait()
            plsc.subcore_barrier()

            for i in range(10):
                pltpu.sync_copy(acc.at[pl.ds(roff + i * 64, 64)],
                                buf.at[pl.ds(i * 64, 64)])
            pltpu.sync_copy(buf.at[pl.ds(0, 640)], out_ref.at[pl.ds(roff, 640)])
            plsc.subcore_barrier()

        @pl.when(c == 0)
        def _():
            quarter(u0_ref, a0_ref)
            quarter(u1_ref, a1_ref)

        @pl.when(c == 1)
        def _():
            quarter(u2_ref, a2_ref)
            quarter(u3_ref, a3_ref)

    f = pl.kernel(
        body,
        out_type=[jax.ShapeDtypeStruct((NP, Q), jnp.float32)] * 4,
        mesh=_mesh(),
        compiler_params=pltpu.CompilerParams(use_tc_tiling_on_sc=False),
        scratch_types=[
            pltpu.VMEM_SHARED((NP, Q), jnp.float32),
            pltpu.VMEM((_D * _CW, Q), jnp.float32),
            pltpu.VMEM((16, Q), jnp.float32),
            pltpu.VMEM((EP // 16,), jnp.int32),
            pltpu.VMEM((EP // 16,), jnp.int32),
            pltpu.SemaphoreType.DMA,
            pltpu.SemaphoreType.DMA,
            pltpu.SemaphoreType.DMA,
            pltpu.SemaphoreType.DMA,
            pltpu.SemaphoreType.DMA,
            pltpu.SemaphoreType.DMA,
            pltpu.SemaphoreType.DMA,
            pltpu.SemaphoreType.DMA,
            pltpu.SemaphoreType.DMA,
            pltpu.SemaphoreType.DMA,
            pltpu.SemaphoreType.DMA,
        ],
    )
    return f(u0, u1, u2, u3, srcp, dstp)


def _tc_l0(xp, hist, w0p):
    def body(xp_ref, hist_ref, w_ref, u0_ref, u1_ref, u2_ref, u3_ref,
             dinv_ref):
        pid = pl.program_id(0)
        rows = lax.broadcasted_iota(jnp.int32, (BLKN, 1), 0) + pid * BLKN
        dinv = jnp.where(rows < N, lax.rsqrt(hist_ref[...] + 1.0), 0.0)
        dinv_ref[...] = dinv
        xb = xp_ref[...]
        w = w_ref[...]
        outs = (u0_ref, u1_ref, u2_ref, u3_ref)
        for b in range(B):
            hw = jnp.dot(xb[:, b * 8:(b + 1) * 8], w,
                         preferred_element_type=jnp.float32)
            u = hw * dinv
            outs[b // 2][:, (b % 2) * C:(b % 2 + 1) * C] = u

    return pl.pallas_call(
        body,
        grid=(GRID,),
        in_specs=[
            pl.BlockSpec((BLKN, 64), lambda i: (i, 0)),
            pl.BlockSpec((BLKN, 1), lambda i: (i, 0)),
            pl.BlockSpec((8, C), lambda i: (0, 0)),
        ],
        out_specs=[pl.BlockSpec((BLKN, Q), lambda i: (i, 0))] * 4
        + [pl.BlockSpec((BLKN, 1), lambda i: (i, 0))],
        out_shape=[jax.ShapeDtypeStruct((NP, Q), jnp.float32)] * 4
        + [jax.ShapeDtypeStruct((NP, 1), jnp.float32)],
    )(xp, hist, w0p)


def _tc_mid(aggs, us, dinv, bprev, w):
    def body(a0_ref, a1_ref, a2_ref, a3_ref, u0_ref, u1_ref, u2_ref, u3_ref,
             dinv_ref, b_ref, w_ref, o0_ref, o1_ref, o2_ref, o3_ref):
        dv = dinv_ref[...]
        bb = b_ref[...]
        w = w_ref[...]
        a_refs = (a0_ref, a1_ref, a2_ref, a3_ref)
        u_refs = (u0_ref, u1_ref, u2_ref, u3_ref)
        o_refs = (o0_ref, o1_ref, o2_ref, o3_ref)
        for q in range(4):
            av = a_refs[q][...]
            uv = u_refs[q][...]
            for g in range(2):
                sl = slice(g * C, (g + 1) * C)
                h = jnp.maximum(dv * (av[:, sl] + uv[:, sl]) + bb, 0.0)
                o_refs[q][:, sl] = dv * jnp.dot(
                    h, w, preferred_element_type=jnp.float32)

    blk = pl.BlockSpec((BLKN, Q), lambda i: (i, 0))
    return pl.pallas_call(
        body,
        grid=(GRID,),
        in_specs=[blk] * 8 + [
            pl.BlockSpec((BLKN, 1), lambda i: (i, 0)),
            pl.BlockSpec((1, C), lambda i: (0, 0)),
            pl.BlockSpec((C, C), lambda i: (0, 0)),
        ],
        out_specs=[blk] * 4,
        out_shape=[jax.ShapeDtypeStruct((NP, Q), jnp.float32)] * 4,
    )(*aggs, *us, dinv, bprev, w)


def _tc_head(aggs, us, dinv, b4, wp1, bp1, wp2, bp2):
    def body(a0_ref, a1_ref, a2_ref, a3_ref, u0_ref, u1_ref, u2_ref, u3_ref,
             dinv_ref, b_ref, wp1_ref, bp1_ref, wp2_ref, bp2_ref,
             lg_ref, ps_ref):
        pid = pl.program_id(0)
        rows = lax.broadcasted_iota(jnp.int32, (BLKN, 1), 0) + pid * BLKN
        mask = rows < N
        dv = dinv_ref[...]
        bb = b_ref[...]
        wp1 = wp1_ref[...]
        wp2 = wp2_ref[...]
        a_refs = (a0_ref, a1_ref, a2_ref, a3_ref)
        u_refs = (u0_ref, u1_ref, u2_ref, u3_ref)
        sums = []
        for b in range(B):
            q, g = b // 2, b % 2
            sl = slice(g * C, (g + 1) * C)
            h = jnp.maximum(dv * (a_refs[q][:, sl] + u_refs[q][:, sl]) + bb,
                            0.0)
            z = jnp.maximum(jnp.dot(h, wp1, preferred_element_type=jnp.float32)
                            + bp1_ref[...], 0.0)
            lg = jnp.dot(z, wp2, preferred_element_type=jnp.float32) \
                + bp2_ref[...]
            lg_ref[:, b * 4:(b + 1) * 4] = lg
            hm = jnp.where(mask, h, 0.0)
            sums.append(jnp.sum(hm, axis=0, keepdims=True))
        part = jnp.concatenate(sums, axis=0)

        @pl.when(pid == 0)
        def _():
            ps_ref[...] = jnp.zeros_like(ps_ref)

        ps_ref[...] += part

    blk = pl.BlockSpec((BLKN, Q), lambda i: (i, 0))
    return pl.pallas_call(
        body,
        grid=(GRID,),
        in_specs=[blk] * 8 + [
            pl.BlockSpec((BLKN, 1), lambda i: (i, 0)),
            pl.BlockSpec((1, C), lambda i: (0, 0)),
            pl.BlockSpec((C, 128), lambda i: (0, 0)),
            pl.BlockSpec((1, 128), lambda i: (0, 0)),
            pl.BlockSpec((128, 4), lambda i: (0, 0)),
            pl.BlockSpec((1, 4), lambda i: (0, 0)),
        ],
        out_specs=[
            pl.BlockSpec((BLKN, 32), lambda i: (i, 0)),
            pl.BlockSpec((B, C), lambda i: (0, 0)),
        ],
        out_shape=[
            jax.ShapeDtypeStruct((NP, 32), jnp.float32),
            jax.ShapeDtypeStruct((B, C), jnp.float32),
        ],
    )(*aggs, *us, dinv, b4, wp1, bp1, wp2, bp2)


def _tc_final(lp, ps, wv1, bv1, wv2, bv2, wv3, bv3):
    def body(lp_ref, ps_ref, wv1_ref, bv1_ref, wv2_ref, bv2_ref,
             wv3_ref, bv3_ref, pol_ref, val_ref):
        lp = lp_ref[...]
        m = jnp.max(lp, axis=1, keepdims=True)
        e = jnp.exp(lp - m)
        pol_ref[...] = e / jnp.sum(e, axis=1, keepdims=True)
        pooled = ps_ref[...] * (1.0 / N)
        v = jnp.maximum(jnp.dot(pooled, wv1_ref[...],
                                preferred_element_type=jnp.float32)
                        + bv1_ref[...], 0.0)
        v = jnp.maximum(jnp.dot(v, wv2_ref[...],
                                preferred_element_type=jnp.float32)
                        + bv2_ref[...], 0.0)
        val_ref[...] = jnp.tanh(jnp.dot(v, wv3_ref[...],
                                        preferred_element_type=jnp.float32)
                                + bv3_ref[...])

    return pl.pallas_call(
        body,
        out_shape=[
            jax.ShapeDtypeStruct((B, N * 4), jnp.float32),
            jax.ShapeDtypeStruct((B, 1), jnp.float32),
        ],
    )(lp, ps, wv1, bv1, wv2, bv2, wv3, bv3)


def kernel(x, edge_index, W0, b0, W1, b1, W2, b2, W3, b3, W4, b4,
           Wp1, bp1, Wp2, bp2, Wv1, bv1, Wv2, bv2, Wv3, bv3):
    xt = jnp.transpose(x, (1, 0, 2))
    xt = jnp.pad(xt, ((0, NP - N), (0, 0), (0, 3)))
    xp = xt.reshape(NP, B * 8)
    w0p = jnp.pad(W0, ((0, 3), (0, 0)))

    src = edge_index[0]
    dst = edge_index[1]
    pad = jnp.full((EP - E,), NP - 1, dtype=src.dtype)
    srcp = jnp.concatenate([src, pad])
    dstp = jnp.concatenate([dst, pad])

    hist = _sc_degree(dstp).reshape(NP, 1)
    *us, dinv = _tc_l0(xp, hist, w0p)

    ws = [W1, W2, W3, W4]
    bs = [b0, b1, b2, b3]
    for i in range(4):
        aggs = _sc_segsum(*us, srcp, dstp)
        us = _tc_mid(aggs, us, dinv, bs[i].reshape(1, C), ws[i])
    aggs = _sc_segsum(*us, srcp, dstp)

    lg, ps = _tc_head(aggs, us, dinv, b4.reshape(1, C),
                      Wp1, bp1.reshape(1, 128), Wp2, bp2.reshape(1, 4))

    lp = lg[:N].reshape(N, B, 4).transpose(1, 0, 2).reshape(B, N * 4)

    policy, value = _tc_final(lp, ps, Wv1, bv1.reshape(1, 256),
                              Wv2, bv2.reshape(1, 64), Wv3, bv3.reshape(1, 1))
    return policy, value


# speedup vs baseline: 1.0015x; 1.0015x over previous
import functools

import jax
import jax.numpy as jnp
from jax import lax
from jax.experimental import pallas as pl
from jax.experimental.pallas import tpu as pltpu
from jax.experimental.pallas import tpu_sc as plsc

N = 10000
NP = 10240
B = 8
C = 32
E = 160000
EP = 163840
CHUNKS = EP // 16 // 128
BLKN = 1024
GRID = NP // BLKN
Q = 64


@functools.cache
def _mesh():
    return plsc.VectorSubcoreMesh(core_axis_name="c", subcore_axis_name="s")


def _sc_degree(dstp):
    def body(dst_ref, hist_ref, acc, zbuf, didx, ones, tb, isem, ssem):
        c = lax.axis_index("c")
        s = lax.axis_index("s")
        ebase = pl.multiple_of(s * (EP // 16), 128)
        hidx = pltpu.async_copy(dst_ref.at[pl.ds(ebase, EP // 16)], didx,
                                isem)
        for i in range(8):
            zbuf[pl.ds(i * 16, 16)] = jnp.zeros((16,), jnp.float32)
            ones[pl.ds(i * 16, 16)] = jnp.ones((16,), jnp.float32)
        for i in range(5):
            off = pl.multiple_of(s * 640 + i * 128, 128)
            pltpu.sync_copy(zbuf, acc.at[pl.ds(off, 128)])
        hidx.wait()
        plsc.subcore_barrier()

        hs = []
        for k in range(CHUNKS):
            off = pl.multiple_of(k * 128, 128)
            hs.append(pltpu.async_copy(ones, acc.at[didx.at[pl.ds(off, 128)]],
                                       ssem, add=True))
        for h in hs:
            h.wait()
        plsc.subcore_barrier()

        off = pl.multiple_of(c * (NP // 2) + s * 320, 64)
        pltpu.sync_copy(acc.at[pl.ds(off, 320)], tb)
        pltpu.sync_copy(tb, hist_ref.at[pl.ds(off, 320)])

    f = pl.kernel(
        body,
        out_type=jax.ShapeDtypeStruct((NP,), jnp.float32),
        mesh=_mesh(),
        scratch_types=[
            pltpu.VMEM_SHARED((NP,), jnp.float32),
            pltpu.VMEM((128,), jnp.float32),
            pltpu.VMEM((EP // 16,), jnp.int32),
            pltpu.VMEM((128,), jnp.float32),
            pltpu.VMEM((320,), jnp.float32),
            pltpu.SemaphoreType.DMA,
            pltpu.SemaphoreType.DMA,
        ],
    )
    return f(dstp)


_D = 4
_LAG = 3
_CW = 256
_NCH = EP // 16 // _CW


def _sc_segsum(u0, u1, u2, u3, srcp, dstp):
    def body(u0_ref, u1_ref, u2_ref, u3_ref, src_ref, dst_ref,
             a0_ref, a1_ref, a2_ref, a3_ref,
             acc, buf, zbuf, sidx, didx, isem,
             g0, g1, g2, g3, g4, s0, s1, s2, s3, s4):
        c = lax.axis_index("c")
        s = lax.axis_index("s")
        ebase = pl.multiple_of(s * (EP // 16), 128)
        roff = pl.multiple_of(s * 640, 64)
        gsems = (g0, g1, g2, g3, g4)
        ssems = (s0, s1, s2, s3, s4)

        hsrc = pltpu.async_copy(src_ref.at[pl.ds(ebase, EP // 16)], sidx,
                                isem)
        hdst = pltpu.async_copy(dst_ref.at[pl.ds(ebase, EP // 16)], didx,
                                isem)

        for j in range(Q // 16):
            for i in range(16):
                zbuf[i, pl.ds(j * 16, 16)] = jnp.zeros((16,), jnp.float32)
        hsrc.wait()
        hdst.wait()

        def quarter(u_ref, out_ref):
            for i in range(40):
                off = pl.multiple_of(s * 640 + i * 16, 16)
                pltpu.sync_copy(zbuf, acc.at[pl.ds(off, 16)])
            plsc.subcore_barrier()

            gh = [None] * _NCH
            sh = [None] * _NCH

            def gissue(k):
                slot = k % _D
                off = pl.multiple_of(k * _CW, 128)
                return pltpu.async_copy(
                    u_ref.at[sidx.at[pl.ds(off, _CW)]],
                    buf.at[pl.ds(slot * _CW, _CW)], gsems[slot])

            def sissue(k):
                slot = k % _D
                off = pl.multiple_of(k * _CW, 128)
                return pltpu.async_copy(
                    buf.at[pl.ds(slot * _CW, _CW)],
                    acc.at[didx.at[pl.ds(off, _CW)]], ssems[slot], add=True)

            for k in range(_NCH):
                if k >= _D:
                    sh[k - _D].wait()
                gh[k] = gissue(k)
                j = k - _LAG
                if j >= 0:
                    gh[j].wait()
                    sh[j] = sissue(j)
            for j in range(_NCH - _LAG, _NCH):
                gh[j].wait()
                sh[j] = sissue(j)
            for j in range(_NCH - _D, _NCH):
                sh[j].wait()
            plsc.subcore_barrier()

            for i in range(10):
                pltpu.sync_copy(acc.at[pl.ds(roff + i * 64, 64)],
                                buf.at[pl.ds(i * 64, 64)])
            pltpu.sync_copy(buf.at[pl.ds(0, 640)], out_ref.at[pl.ds(roff, 640)])
            plsc.subcore_barrier()

        @pl.when(c == 0)
        def _():
            quarter(u0_ref, a0_ref)
            quarter(u1_ref, a1_ref)

        @pl.when(c == 1)
        def _():
            quarter(u2_ref, a2_ref)
            quarter(u3_ref, a3_ref)

    f = pl.kernel(
        body,
        out_type=[jax.ShapeDtypeStruct((NP, Q), jnp.float32)] * 4,
        mesh=_mesh(),
        compiler_params=pltpu.CompilerParams(use_tc_tiling_on_sc=False),
        scratch_types=[
            pltpu.VMEM_SHARED((NP, Q), jnp.float32),
            pltpu.VMEM((_D * _CW, Q), jnp.float32),
            pltpu.VMEM((16, Q), jnp.float32),
            pltpu.VMEM((EP // 16,), jnp.int32),
            pltpu.VMEM((EP // 16,), jnp.int32),
            pltpu.SemaphoreType.DMA,
            pltpu.SemaphoreType.DMA,
            pltpu.SemaphoreType.DMA,
            pltpu.SemaphoreType.DMA,
            pltpu.SemaphoreType.DMA,
            pltpu.SemaphoreType.DMA,
            pltpu.SemaphoreType.DMA,
            pltpu.SemaphoreType.DMA,
            pltpu.SemaphoreType.DMA,
            pltpu.SemaphoreType.DMA,
            pltpu.SemaphoreType.DMA,
        ],
    )
    return f(u0, u1, u2, u3, srcp, dstp)


def _tc_l0(xp, hist, w0p):
    def body(xp_ref, hist_ref, w_ref, u0_ref, u1_ref, u2_ref, u3_ref,
             dinv_ref):
        pid = pl.program_id(0)
        rows = lax.broadcasted_iota(jnp.int32, (BLKN, 1), 0) + pid * BLKN
        dinv = jnp.where(rows < N, lax.rsqrt(hist_ref[...] + 1.0), 0.0)
        dinv_ref[...] = dinv
        xb = xp_ref[...]
        w = w_ref[...]
        outs = (u0_ref, u1_ref, u2_ref, u3_ref)
        for b in range(B):
            hw = jnp.dot(xb[:, b * 8:(b + 1) * 8], w,
                         preferred_element_type=jnp.float32)
            u = hw * dinv
            outs[b // 2][:, (b % 2) * C:(b % 2 + 1) * C] = u

    return pl.pallas_call(
        body,
        grid=(GRID,),
        in_specs=[
            pl.BlockSpec((BLKN, 64), lambda i: (i, 0)),
            pl.BlockSpec((BLKN, 1), lambda i: (i, 0)),
            pl.BlockSpec((8, C), lambda i: (0, 0)),
        ],
        out_specs=[pl.BlockSpec((BLKN, Q), lambda i: (i, 0))] * 4
        + [pl.BlockSpec((BLKN, 1), lambda i: (i, 0))],
        out_shape=[jax.ShapeDtypeStruct((NP, Q), jnp.float32)] * 4
        + [jax.ShapeDtypeStruct((NP, 1), jnp.float32)],
    )(xp, hist, w0p)


def _tc_mid(aggs, us, dinv, bprev, w):
    def body(a0_ref, a1_ref, a2_ref, a3_ref, u0_ref, u1_ref, u2_ref, u3_ref,
             dinv_ref, b_ref, w_ref, o0_ref, o1_ref, o2_ref, o3_ref):
        dv = dinv_ref[...]
        bb = b_ref[...]
        w = w_ref[...]
        a_refs = (a0_ref, a1_ref, a2_ref, a3_ref)
        u_refs = (u0_ref, u1_ref, u2_ref, u3_ref)
        o_refs = (o0_ref, o1_ref, o2_ref, o3_ref)
        for q in range(4):
            av = a_refs[q][...]
            uv = u_refs[q][...]
            for g in range(2):
                sl = slice(g * C, (g + 1) * C)
                h = jnp.maximum(dv * (av[:, sl] + uv[:, sl]) + bb, 0.0)
                o_refs[q][:, sl] = dv * jnp.dot(
                    h, w, preferred_element_type=jnp.float32)

    blk = pl.BlockSpec((BLKN, Q), lambda i: (i, 0))
    return pl.pallas_call(
        body,
        grid=(GRID,),
        in_specs=[blk] * 8 + [
            pl.BlockSpec((BLKN, 1), lambda i: (i, 0)),
            pl.BlockSpec((1, C), lambda i: (0, 0)),
            pl.BlockSpec((C, C), lambda i: (0, 0)),
        ],
        out_specs=[blk] * 4,
        out_shape=[jax.ShapeDtypeStruct((NP, Q), jnp.float32)] * 4,
    )(*aggs, *us, dinv, bprev, w)


def _tc_head(aggs, us, dinv, b4, wp1, bp1, wp2, bp2):
    def body(a0_ref, a1_ref, a2_ref, a3_ref, u0_ref, u1_ref, u2_ref, u3_ref,
             dinv_ref, b_ref, wp1_ref, bp1_ref, wp2_ref, bp2_ref,
             lg_ref, ps_ref):
        pid = pl.program_id(0)
        rows = lax.broadcasted_iota(jnp.int32, (BLKN, 1), 0) + pid * BLKN
        mask = rows < N
        dv = dinv_ref[...]
        bb = b_ref[...]
        wp1 = wp1_ref[...]
        wp2 = wp2_ref[...]
        a_refs = (a0_ref, a1_ref, a2_ref, a3_ref)
        u_refs = (u0_ref, u1_ref, u2_ref, u3_ref)
        sums = []
        for b in range(B):
            q, g = b // 2, b % 2
            sl = slice(g * C, (g + 1) * C)
            h = jnp.maximum(dv * (a_refs[q][:, sl] + u_refs[q][:, sl]) + bb,
                            0.0)
            z = jnp.maximum(jnp.dot(h, wp1, preferred_element_type=jnp.float32)
                            + bp1_ref[...], 0.0)
            lg = jnp.dot(z, wp2, preferred_element_type=jnp.float32) \
                + bp2_ref[...]
            lg_ref[:, b * 4:(b + 1) * 4] = lg
            hm = jnp.where(mask, h, 0.0)
            sums.append(jnp.sum(hm, axis=0, keepdims=True))
        part = jnp.concatenate(sums, axis=0)

        @pl.when(pid == 0)
        def _():
            ps_ref[...] = jnp.zeros_like(ps_ref)

        ps_ref[...] += part

    blk = pl.BlockSpec((BLKN, Q), lambda i: (i, 0))
    return pl.pallas_call(
        body,
        grid=(GRID,),
        in_specs=[blk] * 8 + [
            pl.BlockSpec((BLKN, 1), lambda i: (i, 0)),
            pl.BlockSpec((1, C), lambda i: (0, 0)),
            pl.BlockSpec((C, 128), lambda i: (0, 0)),
            pl.BlockSpec((1, 128), lambda i: (0, 0)),
            pl.BlockSpec((128, 4), lambda i: (0, 0)),
            pl.BlockSpec((1, 4), lambda i: (0, 0)),
        ],
        out_specs=[
            pl.BlockSpec((BLKN, 32), lambda i: (i, 0)),
            pl.BlockSpec((B, C), lambda i: (0, 0)),
        ],
        out_shape=[
            jax.ShapeDtypeStruct((NP, 32), jnp.float32),
            jax.ShapeDtypeStruct((B, C), jnp.float32),
        ],
    )(*aggs, *us, dinv, b4, wp1, bp1, wp2, bp2)


def _tc_final(lp, ps, wv1, bv1, wv2, bv2, wv3, bv3):
    def body(lp_ref, ps_ref, wv1_ref, bv1_ref, wv2_ref, bv2_ref,
             wv3_ref, bv3_ref, pol_ref, val_ref):
        lp = lp_ref[...]
        m = jnp.max(lp, axis=1, keepdims=True)
        e = jnp.exp(lp - m)
        pol_ref[...] = e / jnp.sum(e, axis=1, keepdims=True)
        pooled = ps_ref[...] * (1.0 / N)
        v = jnp.maximum(jnp.dot(pooled, wv1_ref[...],
                                preferred_element_type=jnp.float32)
                        + bv1_ref[...], 0.0)
        v = jnp.maximum(jnp.dot(v, wv2_ref[...],
                                preferred_element_type=jnp.float32)
                        + bv2_ref[...], 0.0)
        val_ref[...] = jnp.tanh(jnp.dot(v, wv3_ref[...],
                                        preferred_element_type=jnp.float32)
                                + bv3_ref[...])

    return pl.pallas_call(
        body,
        out_shape=[
            jax.ShapeDtypeStruct((B, N * 4), jnp.float32),
            jax.ShapeDtypeStruct((B, 1), jnp.float32),
        ],
    )(lp, ps, wv1, bv1, wv2, bv2, wv3, bv3)


def kernel(x, edge_index, W0, b0, W1, b1, W2, b2, W3, b3, W4, b4,
           Wp1, bp1, Wp2, bp2, Wv1, bv1, Wv2, bv2, Wv3, bv3):
    xt = jnp.transpose(x, (1, 0, 2))
    xt = jnp.pad(xt, ((0, NP - N), (0, 0), (0, 3)))
    xp = xt.reshape(NP, B * 8)
    w0p = jnp.pad(W0, ((0, 3), (0, 0)))

    src = edge_index[0]
    dst = edge_index[1]
    pad = jnp.full((EP - E,), NP - 1, dtype=src.dtype)
    srcp = jnp.concatenate([src, pad])
    dstp = jnp.concatenate([dst, pad])

    hist = _sc_degree(dstp).reshape(NP, 1)
    *us, dinv = _tc_l0(xp, hist, w0p)

    ws = [W1, W2, W3, W4]
    bs = [b0, b1, b2, b3]
    for i in range(4):
        aggs = _sc_segsum(*us, srcp, dstp)
        us = _tc_mid(aggs, us, dinv, bs[i].reshape(1, C), ws[i])
    aggs = _sc_segsum(*us, srcp, dstp)

    lg, ps = _tc_head(aggs, us, dinv, b4.reshape(1, C),
                      Wp1, bp1.reshape(1, 128), Wp2, bp2.reshape(1, 4))

    lp = lg[:N].reshape(N, B, 4).transpose(1, 0, 2).reshape(B, N * 4)

    policy, value = _tc_final(lp, ps, Wv1, bv1.reshape(1, 256),
                              Wv2, bv2.reshape(1, 64), Wv3, bv3.reshape(1, 1))
    return policy, value


# speedup vs baseline: 1.6259x; 1.6234x over previous
import functools

import jax
import jax.numpy as jnp
from jax import lax
from jax.experimental import pallas as pl
from jax.experimental.pallas import tpu as pltpu
from jax.experimental.pallas import tpu_sc as plsc

N = 10000
NP = 10240
B = 8
C = 32
E = 160000
EP = 163840
CHUNKS = EP // 16 // 128
BLKN = 1024
GRID = NP // BLKN
Q = 64


@functools.cache
def _mesh():
    return plsc.VectorSubcoreMesh(core_axis_name="c", subcore_axis_name="s")


def _sc_degree(dstp):
    def body(dst_ref, hist_ref, acc, zbuf, didx, ones, tb, isem, ssem):
        c = lax.axis_index("c")
        s = lax.axis_index("s")
        ebase = pl.multiple_of(s * (EP // 16), 128)
        hidx = pltpu.async_copy(dst_ref.at[pl.ds(ebase, EP // 16)], didx,
                                isem)
        for i in range(8):
            zbuf[pl.ds(i * 16, 16)] = jnp.zeros((16,), jnp.float32)
            ones[pl.ds(i * 16, 16)] = jnp.ones((16,), jnp.float32)
        for i in range(5):
            off = pl.multiple_of(s * 640 + i * 128, 128)
            pltpu.sync_copy(zbuf, acc.at[pl.ds(off, 128)])
        hidx.wait()
        plsc.subcore_barrier()

        hs = []
        for k in range(CHUNKS):
            off = pl.multiple_of(k * 128, 128)
            hs.append(pltpu.async_copy(ones, acc.at[didx.at[pl.ds(off, 128)]],
                                       ssem, add=True))
        for h in hs:
            h.wait()
        plsc.subcore_barrier()

        off = pl.multiple_of(c * (NP // 2) + s * 320, 64)
        pltpu.sync_copy(acc.at[pl.ds(off, 320)], tb)
        pltpu.sync_copy(tb, hist_ref.at[pl.ds(off, 320)])

    f = pl.kernel(
        body,
        out_type=jax.ShapeDtypeStruct((NP,), jnp.float32),
        mesh=_mesh(),
        scratch_types=[
            pltpu.VMEM_SHARED((NP,), jnp.float32),
            pltpu.VMEM((128,), jnp.float32),
            pltpu.VMEM((EP // 16,), jnp.int32),
            pltpu.VMEM((128,), jnp.float32),
            pltpu.VMEM((320,), jnp.float32),
            pltpu.SemaphoreType.DMA,
            pltpu.SemaphoreType.DMA,
        ],
    )
    return f(dstp)


_D = 4
_LAG = 3
_CW = 256
_NCH = EP // 16 // _CW


def _sc_segsum(u0, u1, u2, u3, srcp, dstp):
    def body(u0_ref, u1_ref, u2_ref, u3_ref, src_ref, dst_ref,
             a0_ref, a1_ref, a2_ref, a3_ref,
             usp, acc, buf, zbuf, sidx, didx, isem,
             g0, g1, g2, g3, g4, s0, s1, s2, s3, s4):
        c = lax.axis_index("c")
        s = lax.axis_index("s")
        ebase = pl.multiple_of(s * (EP // 16), 128)
        roff = pl.multiple_of(s * 640, 64)
        gsems = (g0, g1, g2, g3, g4)
        ssems = (s0, s1, s2, s3, s4)
        H = Q // 2

        hsrc = pltpu.async_copy(src_ref.at[pl.ds(ebase, EP // 16)], sidx,
                                isem)
        hdst = pltpu.async_copy(dst_ref.at[pl.ds(ebase, EP // 16)], didx,
                                isem)

        for j in range(H // 16):
            for i in range(64):
                zbuf[i, pl.ds(j * 16, 16)] = jnp.zeros((16,), jnp.float32)
        hsrc.wait()
        hdst.wait()

        def half(u_ref, out_ref, h):
            pltpu.sync_copy(u_ref.at[pl.ds(roff, 640), pl.ds(h * H, H)],
                            buf.at[pl.ds(0, 640)])
            pltpu.sync_copy(buf.at[pl.ds(0, 640)], usp.at[pl.ds(roff, 640)])
            for i in range(10):
                off = pl.multiple_of(s * 640 + i * 64, 64)
                pltpu.sync_copy(zbuf, acc.at[pl.ds(off, 64)])
            plsc.subcore_barrier()

            gh = [None] * _NCH
            sh = [None] * _NCH

            def gissue(k):
                slot = k % _D
                off = pl.multiple_of(k * _CW, 128)
                return pltpu.async_copy(
                    usp.at[sidx.at[pl.ds(off, _CW)]],
                    buf.at[pl.ds(slot * _CW, _CW)], gsems[slot])

            def sissue(k):
                slot = k % _D
                off = pl.multiple_of(k * _CW, 128)
                return pltpu.async_copy(
                    buf.at[pl.ds(slot * _CW, _CW)],
                    acc.at[didx.at[pl.ds(off, _CW)]], ssems[slot], add=True)

            for k in range(_NCH):
                if k >= _D:
                    sh[k - _D].wait()
                gh[k] = gissue(k)
                j = k - _LAG
                if j >= 0:
                    gh[j].wait()
                    sh[j] = sissue(j)
            for j in range(_NCH - _LAG, _NCH):
                gh[j].wait()
                sh[j] = sissue(j)
            for j in range(_NCH - _D, _NCH):
                sh[j].wait()
            plsc.subcore_barrier()

            for i in range(10):
                pltpu.sync_copy(acc.at[pl.ds(roff + i * 64, 64)],
                                buf.at[pl.ds(i * 64, 64)])
            pltpu.sync_copy(buf.at[pl.ds(0, 640)],
                            out_ref.at[pl.ds(roff, 640), pl.ds(h * H, H)])
            plsc.subcore_barrier()

        @pl.when(c == 0)
        def _():
            half(u0_ref, a0_ref, 0)
            half(u0_ref, a0_ref, 1)
            half(u1_ref, a1_ref, 0)
            half(u1_ref, a1_ref, 1)

        @pl.when(c == 1)
        def _():
            half(u2_ref, a2_ref, 0)
            half(u2_ref, a2_ref, 1)
            half(u3_ref, a3_ref, 0)
            half(u3_ref, a3_ref, 1)

    f = pl.kernel(
        body,
        out_type=[jax.ShapeDtypeStruct((NP, Q), jnp.float32)] * 4,
        mesh=_mesh(),
        compiler_params=pltpu.CompilerParams(use_tc_tiling_on_sc=False),
        scratch_types=[
            pltpu.VMEM_SHARED((NP, Q // 2), jnp.float32),
            pltpu.VMEM_SHARED((NP, Q // 2), jnp.float32),
            pltpu.VMEM((_D * _CW, Q // 2), jnp.float32),
            pltpu.VMEM((64, Q // 2), jnp.float32),
            pltpu.VMEM((EP // 16,), jnp.int32),
            pltpu.VMEM((EP // 16,), jnp.int32),
            pltpu.SemaphoreType.DMA,
            pltpu.SemaphoreType.DMA,
            pltpu.SemaphoreType.DMA,
            pltpu.SemaphoreType.DMA,
            pltpu.SemaphoreType.DMA,
            pltpu.SemaphoreType.DMA,
            pltpu.SemaphoreType.DMA,
            pltpu.SemaphoreType.DMA,
            pltpu.SemaphoreType.DMA,
            pltpu.SemaphoreType.DMA,
            pltpu.SemaphoreType.DMA,
        ],
    )
    return f(u0, u1, u2, u3, srcp, dstp)


def _tc_l0(xp, hist, w0p):
    def body(xp_ref, hist_ref, w_ref, u0_ref, u1_ref, u2_ref, u3_ref,
             dinv_ref):
        pid = pl.program_id(0)
        rows = lax.broadcasted_iota(jnp.int32, (BLKN, 1), 0) + pid * BLKN
        dinv = jnp.where(rows < N, lax.rsqrt(hist_ref[...] + 1.0), 0.0)
        dinv_ref[...] = dinv
        xb = xp_ref[...]
        w = w_ref[...]
        outs = (u0_ref, u1_ref, u2_ref, u3_ref)
        for b in range(B):
            hw = jnp.dot(xb[:, b * 8:(b + 1) * 8], w,
                         preferred_element_type=jnp.float32)
            u = hw * dinv
            outs[b // 2][:, (b % 2) * C:(b % 2 + 1) * C] = u

    return pl.pallas_call(
        body,
        grid=(GRID,),
        in_specs=[
            pl.BlockSpec((BLKN, 64), lambda i: (i, 0)),
            pl.BlockSpec((BLKN, 1), lambda i: (i, 0)),
            pl.BlockSpec((8, C), lambda i: (0, 0)),
        ],
        out_specs=[pl.BlockSpec((BLKN, Q), lambda i: (i, 0))] * 4
        + [pl.BlockSpec((BLKN, 1), lambda i: (i, 0))],
        out_shape=[jax.ShapeDtypeStruct((NP, Q), jnp.float32)] * 4
        + [jax.ShapeDtypeStruct((NP, 1), jnp.float32)],
    )(xp, hist, w0p)


def _tc_mid(aggs, us, dinv, bprev, w):
    def body(a0_ref, a1_ref, a2_ref, a3_ref, u0_ref, u1_ref, u2_ref, u3_ref,
             dinv_ref, b_ref, w_ref, o0_ref, o1_ref, o2_ref, o3_ref):
        dv = dinv_ref[...]
        bb = b_ref[...]
        w = w_ref[...]
        a_refs = (a0_ref, a1_ref, a2_ref, a3_ref)
        u_refs = (u0_ref, u1_ref, u2_ref, u3_ref)
        o_refs = (o0_ref, o1_ref, o2_ref, o3_ref)
        for q in range(4):
            av = a_refs[q][...]
            uv = u_refs[q][...]
            for g in range(2):
                sl = slice(g * C, (g + 1) * C)
                h = jnp.maximum(dv * (av[:, sl] + uv[:, sl]) + bb, 0.0)
                o_refs[q][:, sl] = dv * jnp.dot(
                    h, w, preferred_element_type=jnp.float32)

    blk = pl.BlockSpec((BLKN, Q), lambda i: (i, 0))
    return pl.pallas_call(
        body,
        grid=(GRID,),
        in_specs=[blk] * 8 + [
            pl.BlockSpec((BLKN, 1), lambda i: (i, 0)),
            pl.BlockSpec((1, C), lambda i: (0, 0)),
            pl.BlockSpec((C, C), lambda i: (0, 0)),
        ],
        out_specs=[blk] * 4,
        out_shape=[jax.ShapeDtypeStruct((NP, Q), jnp.float32)] * 4,
    )(*aggs, *us, dinv, bprev, w)


def _tc_head(aggs, us, dinv, b4, wp1, bp1, wp2, bp2):
    def body(a0_ref, a1_ref, a2_ref, a3_ref, u0_ref, u1_ref, u2_ref, u3_ref,
             dinv_ref, b_ref, wp1_ref, bp1_ref, wp2_ref, bp2_ref,
             lg_ref, ps_ref):
        pid = pl.program_id(0)
        rows = lax.broadcasted_iota(jnp.int32, (BLKN, 1), 0) + pid * BLKN
        mask = rows < N
        dv = dinv_ref[...]
        bb = b_ref[...]
        wp1 = wp1_ref[...]
        wp2 = wp2_ref[...]
        a_refs = (a0_ref, a1_ref, a2_ref, a3_ref)
        u_refs = (u0_ref, u1_ref, u2_ref, u3_ref)
        sums = []
        for b in range(B):
            q, g = b // 2, b % 2
            sl = slice(g * C, (g + 1) * C)
            h = jnp.maximum(dv * (a_refs[q][:, sl] + u_refs[q][:, sl]) + bb,
                            0.0)
            z = jnp.maximum(jnp.dot(h, wp1, preferred_element_type=jnp.float32)
                            + bp1_ref[...], 0.0)
            lg = jnp.dot(z, wp2, preferred_element_type=jnp.float32) \
                + bp2_ref[...]
            lg_ref[:, b * 4:(b + 1) * 4] = lg
            hm = jnp.where(mask, h, 0.0)
            sums.append(jnp.sum(hm, axis=0, keepdims=True))
        part = jnp.concatenate(sums, axis=0)

        @pl.when(pid == 0)
        def _():
            ps_ref[...] = jnp.zeros_like(ps_ref)

        ps_ref[...] += part

    blk = pl.BlockSpec((BLKN, Q), lambda i: (i, 0))
    return pl.pallas_call(
        body,
        grid=(GRID,),
        in_specs=[blk] * 8 + [
            pl.BlockSpec((BLKN, 1), lambda i: (i, 0)),
            pl.BlockSpec((1, C), lambda i: (0, 0)),
            pl.BlockSpec((C, 128), lambda i: (0, 0)),
            pl.BlockSpec((1, 128), lambda i: (0, 0)),
            pl.BlockSpec((128, 4), lambda i: (0, 0)),
            pl.BlockSpec((1, 4), lambda i: (0, 0)),
        ],
        out_specs=[
            pl.BlockSpec((BLKN, 32), lambda i: (i, 0)),
            pl.BlockSpec((B, C), lambda i: (0, 0)),
        ],
        out_shape=[
            jax.ShapeDtypeStruct((NP, 32), jnp.float32),
            jax.ShapeDtypeStruct((B, C), jnp.float32),
        ],
    )(*aggs, *us, dinv, b4, wp1, bp1, wp2, bp2)


def _tc_final(lp, ps, wv1, bv1, wv2, bv2, wv3, bv3):
    def body(lp_ref, ps_ref, wv1_ref, bv1_ref, wv2_ref, bv2_ref,
             wv3_ref, bv3_ref, pol_ref, val_ref):
        lp = lp_ref[...]
        m = jnp.max(lp, axis=1, keepdims=True)
        e = jnp.exp(lp - m)
        pol_ref[...] = e / jnp.sum(e, axis=1, keepdims=True)
        pooled = ps_ref[...] * (1.0 / N)
        v = jnp.maximum(jnp.dot(pooled, wv1_ref[...],
                                preferred_element_type=jnp.float32)
                        + bv1_ref[...], 0.0)
        v = jnp.maximum(jnp.dot(v, wv2_ref[...],
                                preferred_element_type=jnp.float32)
                        + bv2_ref[...], 0.0)
        val_ref[...] = jnp.tanh(jnp.dot(v, wv3_ref[...],
                                        preferred_element_type=jnp.float32)
                                + bv3_ref[...])

    return pl.pallas_call(
        body,
        out_shape=[
            jax.ShapeDtypeStruct((B, N * 4), jnp.float32),
            jax.ShapeDtypeStruct((B, 1), jnp.float32),
        ],
    )(lp, ps, wv1, bv1, wv2, bv2, wv3, bv3)


def kernel(x, edge_index, W0, b0, W1, b1, W2, b2, W3, b3, W4, b4,
           Wp1, bp1, Wp2, bp2, Wv1, bv1, Wv2, bv2, Wv3, bv3):
    xt = jnp.transpose(x, (1, 0, 2))
    xt = jnp.pad(xt, ((0, NP - N), (0, 0), (0, 3)))
    xp = xt.reshape(NP, B * 8)
    w0p = jnp.pad(W0, ((0, 3), (0, 0)))

    src = edge_index[0]
    dst = edge_index[1]
    pad = jnp.full((EP - E,), NP - 1, dtype=src.dtype)
    srcp = jnp.concatenate([src, pad])
    dstp = jnp.concatenate([dst, pad])

    hist = _sc_degree(dstp).reshape(NP, 1)
    *us, dinv = _tc_l0(xp, hist, w0p)

    ws = [W1, W2, W3, W4]
    bs = [b0, b1, b2, b3]
    for i in range(4):
        aggs = _sc_segsum(*us, srcp, dstp)
        us = _tc_mid(aggs, us, dinv, bs[i].reshape(1, C), ws[i])
    aggs = _sc_segsum(*us, srcp, dstp)

    lg, ps = _tc_head(aggs, us, dinv, b4.reshape(1, C),
                      Wp1, bp1.reshape(1, 128), Wp2, bp2.reshape(1, 4))

    lp = lg[:N].reshape(N, B, 4).transpose(1, 0, 2).reshape(B, N * 4)

    policy, value = _tc_final(lp, ps, Wv1, bv1.reshape(1, 256),
                              Wv2, bv2.reshape(1, 64), Wv3, bv3.reshape(1, 1))
    return policy, value


# speedup vs baseline: 1.6724x; 1.0287x over previous
import functools

import jax
import jax.numpy as jnp
from jax import lax
from jax.experimental import pallas as pl
from jax.experimental.pallas import tpu as pltpu
from jax.experimental.pallas import tpu_sc as plsc

N = 10000
NP = 10240
B = 8
C = 32
E = 160000
EP = 163840
CHUNKS = EP // 16 // 128
BLKN = 1024
GRID = NP // BLKN
Q = 64


@functools.cache
def _mesh():
    return plsc.VectorSubcoreMesh(core_axis_name="c", subcore_axis_name="s")


def _sc_degree(dstp):
    def body(dst_ref, hist_ref, acc, zbuf, didx, ones, tb, isem, ssem):
        c = lax.axis_index("c")
        s = lax.axis_index("s")
        ebase = pl.multiple_of(s * (EP // 16), 128)
        hidx = pltpu.async_copy(dst_ref.at[pl.ds(ebase, EP // 16)], didx,
                                isem)
        for i in range(8):
            zbuf[pl.ds(i * 16, 16)] = jnp.zeros((16,), jnp.float32)
            ones[pl.ds(i * 16, 16)] = jnp.ones((16,), jnp.float32)
        for i in range(5):
            off = pl.multiple_of(s * 640 + i * 128, 128)
            pltpu.sync_copy(zbuf, acc.at[pl.ds(off, 128)])
        hidx.wait()
        plsc.subcore_barrier()

        hs = []
        for k in range(CHUNKS):
            off = pl.multiple_of(k * 128, 128)
            hs.append(pltpu.async_copy(ones, acc.at[didx.at[pl.ds(off, 128)]],
                                       ssem, add=True))
        for h in hs:
            h.wait()
        plsc.subcore_barrier()

        off = pl.multiple_of(c * (NP // 2) + s * 320, 64)
        pltpu.sync_copy(acc.at[pl.ds(off, 320)], tb)
        pltpu.sync_copy(tb, hist_ref.at[pl.ds(off, 320)])

    f = pl.kernel(
        body,
        out_type=jax.ShapeDtypeStruct((NP,), jnp.float32),
        mesh=_mesh(),
        scratch_types=[
            pltpu.VMEM_SHARED((NP,), jnp.float32),
            pltpu.VMEM((128,), jnp.float32),
            pltpu.VMEM((EP // 16,), jnp.int32),
            pltpu.VMEM((128,), jnp.float32),
            pltpu.VMEM((320,), jnp.float32),
            pltpu.SemaphoreType.DMA,
            pltpu.SemaphoreType.DMA,
        ],
    )
    return f(dstp)


_D = 3
_LAG = 2
_CW = 128
_NCH = EP // 16 // _CW


def _sc_segsum(u0, u1, u2, u3, srcp, dstp):
    def body(u0_ref, u1_ref, u2_ref, u3_ref, src_ref, dst_ref,
             a0_ref, a1_ref, a2_ref, a3_ref,
             usp, acc, buf, zbuf, sidx, didx, isem,
             g0, g1, g2, g3, g4, s0, s1, s2, s3, s4):
        c = lax.axis_index("c")
        s = lax.axis_index("s")
        ebase = pl.multiple_of(s * (EP // 16), 128)
        roff = pl.multiple_of(s * 640, 64)
        gsems = (g0, g1, g2, g3, g4)
        ssems = (s0, s1, s2, s3, s4)
        hsrc = pltpu.async_copy(src_ref.at[pl.ds(ebase, EP // 16)], sidx,
                                isem)
        hdst = pltpu.async_copy(dst_ref.at[pl.ds(ebase, EP // 16)], didx,
                                isem)

        for j in range(Q // 16):
            for i in range(32):
                zbuf[i, pl.ds(j * 16, 16)] = jnp.zeros((16,), jnp.float32)
        hsrc.wait()
        hdst.wait()

        def quarter(u_ref, out_ref):
            for t in range(2):
                toff = pl.multiple_of(roff + t * 320, 64)
                pltpu.sync_copy(u_ref.at[pl.ds(toff, 320)],
                                buf.at[pl.ds(0, 320)])
                pltpu.sync_copy(buf.at[pl.ds(0, 320)],
                                usp.at[pl.ds(toff, 320)])
            for i in range(20):
                off = pl.multiple_of(s * 640 + i * 32, 32)
                pltpu.sync_copy(zbuf, acc.at[pl.ds(off, 32)])
            plsc.subcore_barrier()

            gh = [None] * _NCH
            sh = [None] * _NCH

            def gissue(k):
                slot = k % _D
                off = pl.multiple_of(k * _CW, 128)
                return pltpu.async_copy(
                    usp.at[sidx.at[pl.ds(off, _CW)]],
                    buf.at[pl.ds(slot * _CW, _CW)], gsems[slot])

            def sissue(k):
                slot = k % _D
                off = pl.multiple_of(k * _CW, 128)
                return pltpu.async_copy(
                    buf.at[pl.ds(slot * _CW, _CW)],
                    acc.at[didx.at[pl.ds(off, _CW)]], ssems[slot], add=True)

            for k in range(_NCH):
                if k >= _D:
                    sh[k - _D].wait()
                gh[k] = gissue(k)
                j = k - _LAG
                if j >= 0:
                    gh[j].wait()
                    sh[j] = sissue(j)
            for j in range(_NCH - _LAG, _NCH):
                gh[j].wait()
                sh[j] = sissue(j)
            for j in range(_NCH - _D, _NCH):
                sh[j].wait()
            plsc.subcore_barrier()

            for t in range(2):
                toff = pl.multiple_of(roff + t * 320, 64)
                for i in range(5):
                    pltpu.sync_copy(acc.at[pl.ds(toff + i * 64, 64)],
                                    buf.at[pl.ds(i * 64, 64)])
                pltpu.sync_copy(buf.at[pl.ds(0, 320)],
                                out_ref.at[pl.ds(toff, 320)])
            plsc.subcore_barrier()

        @pl.when(c == 0)
        def _():
            quarter(u0_ref, a0_ref)
            quarter(u1_ref, a1_ref)

        @pl.when(c == 1)
        def _():
            quarter(u2_ref, a2_ref)
            quarter(u3_ref, a3_ref)

    f = pl.kernel(
        body,
        out_type=[jax.ShapeDtypeStruct((NP, Q), jnp.float32)] * 4,
        mesh=_mesh(),
        compiler_params=pltpu.CompilerParams(use_tc_tiling_on_sc=False),
        scratch_types=[
            pltpu.VMEM_SHARED((NP, Q), jnp.float32),
            pltpu.VMEM_SHARED((NP, Q), jnp.float32),
            pltpu.VMEM((_D * _CW, Q), jnp.float32),
            pltpu.VMEM((32, Q), jnp.float32),
            pltpu.VMEM((EP // 16,), jnp.int32),
            pltpu.VMEM((EP // 16,), jnp.int32),
            pltpu.SemaphoreType.DMA,
            pltpu.SemaphoreType.DMA,
            pltpu.SemaphoreType.DMA,
            pltpu.SemaphoreType.DMA,
            pltpu.SemaphoreType.DMA,
            pltpu.SemaphoreType.DMA,
            pltpu.SemaphoreType.DMA,
            pltpu.SemaphoreType.DMA,
            pltpu.SemaphoreType.DMA,
            pltpu.SemaphoreType.DMA,
            pltpu.SemaphoreType.DMA,
        ],
    )
    return f(u0, u1, u2, u3, srcp, dstp)


def _tc_l0(xp, hist, w0p):
    def body(xp_ref, hist_ref, w_ref, u0_ref, u1_ref, u2_ref, u3_ref,
             dinv_ref):
        pid = pl.program_id(0)
        rows = lax.broadcasted_iota(jnp.int32, (BLKN, 1), 0) + pid * BLKN
        dinv = jnp.where(rows < N, lax.rsqrt(hist_ref[...] + 1.0), 0.0)
        dinv_ref[...] = dinv
        xb = xp_ref[...]
        w = w_ref[...]
        outs = (u0_ref, u1_ref, u2_ref, u3_ref)
        for b in range(B):
            hw = jnp.dot(xb[:, b * 8:(b + 1) * 8], w,
                         preferred_element_type=jnp.float32)
            u = hw * dinv
            outs[b // 2][:, (b % 2) * C:(b % 2 + 1) * C] = u

    return pl.pallas_call(
        body,
        grid=(GRID,),
        in_specs=[
            pl.BlockSpec((BLKN, 64), lambda i: (i, 0)),
            pl.BlockSpec((BLKN, 1), lambda i: (i, 0)),
            pl.BlockSpec((8, C), lambda i: (0, 0)),
        ],
        out_specs=[pl.BlockSpec((BLKN, Q), lambda i: (i, 0))] * 4
        + [pl.BlockSpec((BLKN, 1), lambda i: (i, 0))],
        out_shape=[jax.ShapeDtypeStruct((NP, Q), jnp.float32)] * 4
        + [jax.ShapeDtypeStruct((NP, 1), jnp.float32)],
    )(xp, hist, w0p)


def _tc_mid(aggs, us, dinv, bprev, w):
    def body(a0_ref, a1_ref, a2_ref, a3_ref, u0_ref, u1_ref, u2_ref, u3_ref,
             dinv_ref, b_ref, w_ref, o0_ref, o1_ref, o2_ref, o3_ref):
        dv = dinv_ref[...]
        bb = b_ref[...]
        w = w_ref[...]
        a_refs = (a0_ref, a1_ref, a2_ref, a3_ref)
        u_refs = (u0_ref, u1_ref, u2_ref, u3_ref)
        o_refs = (o0_ref, o1_ref, o2_ref, o3_ref)
        for q in range(4):
            av = a_refs[q][...]
            uv = u_refs[q][...]
            for g in range(2):
                sl = slice(g * C, (g + 1) * C)
                h = jnp.maximum(dv * (av[:, sl] + uv[:, sl]) + bb, 0.0)
                o_refs[q][:, sl] = dv * jnp.dot(
                    h, w, preferred_element_type=jnp.float32)

    blk = pl.BlockSpec((BLKN, Q), lambda i: (i, 0))
    return pl.pallas_call(
        body,
        grid=(GRID,),
        in_specs=[blk] * 8 + [
            pl.BlockSpec((BLKN, 1), lambda i: (i, 0)),
            pl.BlockSpec((1, C), lambda i: (0, 0)),
            pl.BlockSpec((C, C), lambda i: (0, 0)),
        ],
        out_specs=[blk] * 4,
        out_shape=[jax.ShapeDtypeStruct((NP, Q), jnp.float32)] * 4,
    )(*aggs, *us, dinv, bprev, w)


def _tc_head(aggs, us, dinv, b4, wp1, bp1, wp2, bp2):
    def body(a0_ref, a1_ref, a2_ref, a3_ref, u0_ref, u1_ref, u2_ref, u3_ref,
             dinv_ref, b_ref, wp1_ref, bp1_ref, wp2_ref, bp2_ref,
             lg_ref, ps_ref):
        pid = pl.program_id(0)
        rows = lax.broadcasted_iota(jnp.int32, (BLKN, 1), 0) + pid * BLKN
        mask = rows < N
        dv = dinv_ref[...]
        bb = b_ref[...]
        wp1 = wp1_ref[...]
        wp2 = wp2_ref[...]
        a_refs = (a0_ref, a1_ref, a2_ref, a3_ref)
        u_refs = (u0_ref, u1_ref, u2_ref, u3_ref)
        sums = []
        for b in range(B):
            q, g = b // 2, b % 2
            sl = slice(g * C, (g + 1) * C)
            h = jnp.maximum(dv * (a_refs[q][:, sl] + u_refs[q][:, sl]) + bb,
                            0.0)
            z = jnp.maximum(jnp.dot(h, wp1, preferred_element_type=jnp.float32)
                            + bp1_ref[...], 0.0)
            lg = jnp.dot(z, wp2, preferred_element_type=jnp.float32) \
                + bp2_ref[...]
            lg_ref[:, b * 4:(b + 1) * 4] = lg
            hm = jnp.where(mask, h, 0.0)
            sums.append(jnp.sum(hm, axis=0, keepdims=True))
        part = jnp.concatenate(sums, axis=0)

        @pl.when(pid == 0)
        def _():
            ps_ref[...] = jnp.zeros_like(ps_ref)

        ps_ref[...] += part

    blk = pl.BlockSpec((BLKN, Q), lambda i: (i, 0))
    return pl.pallas_call(
        body,
        grid=(GRID,),
        in_specs=[blk] * 8 + [
            pl.BlockSpec((BLKN, 1), lambda i: (i, 0)),
            pl.BlockSpec((1, C), lambda i: (0, 0)),
            pl.BlockSpec((C, 128), lambda i: (0, 0)),
            pl.BlockSpec((1, 128), lambda i: (0, 0)),
            pl.BlockSpec((128, 4), lambda i: (0, 0)),
            pl.BlockSpec((1, 4), lambda i: (0, 0)),
        ],
        out_specs=[
            pl.BlockSpec((BLKN, 32), lambda i: (i, 0)),
            pl.BlockSpec((B, C), lambda i: (0, 0)),
        ],
        out_shape=[
            jax.ShapeDtypeStruct((NP, 32), jnp.float32),
            jax.ShapeDtypeStruct((B, C), jnp.float32),
        ],
    )(*aggs, *us, dinv, b4, wp1, bp1, wp2, bp2)


def _tc_final(lp, ps, wv1, bv1, wv2, bv2, wv3, bv3):
    def body(lp_ref, ps_ref, wv1_ref, bv1_ref, wv2_ref, bv2_ref,
             wv3_ref, bv3_ref, pol_ref, val_ref):
        lp = lp_ref[...]
        m = jnp.max(lp, axis=1, keepdims=True)
        e = jnp.exp(lp - m)
        pol_ref[...] = e / jnp.sum(e, axis=1, keepdims=True)
        pooled = ps_ref[...] * (1.0 / N)
        v = jnp.maximum(jnp.dot(pooled, wv1_ref[...],
                                preferred_element_type=jnp.float32)
                        + bv1_ref[...], 0.0)
        v = jnp.maximum(jnp.dot(v, wv2_ref[...],
                                preferred_element_type=jnp.float32)
                        + bv2_ref[...], 0.0)
        val_ref[...] = jnp.tanh(jnp.dot(v, wv3_ref[...],
                                        preferred_element_type=jnp.float32)
                                + bv3_ref[...])

    return pl.pallas_call(
        body,
        out_shape=[
            jax.ShapeDtypeStruct((B, N * 4), jnp.float32),
            jax.ShapeDtypeStruct((B, 1), jnp.float32),
        ],
    )(lp, ps, wv1, bv1, wv2, bv2, wv3, bv3)


def kernel(x, edge_index, W0, b0, W1, b1, W2, b2, W3, b3, W4, b4,
           Wp1, bp1, Wp2, bp2, Wv1, bv1, Wv2, bv2, Wv3, bv3):
    xt = jnp.transpose(x, (1, 0, 2))
    xt = jnp.pad(xt, ((0, NP - N), (0, 0), (0, 3)))
    xp = xt.reshape(NP, B * 8)
    w0p = jnp.pad(W0, ((0, 3), (0, 0)))

    src = edge_index[0]
    dst = edge_index[1]
    pad = jnp.full((EP - E,), NP - 1, dtype=src.dtype)
    srcp = jnp.concatenate([src, pad])
    dstp = jnp.concatenate([dst, pad])

    hist = _sc_degree(dstp).reshape(NP, 1)
    *us, dinv = _tc_l0(xp, hist, w0p)

    ws = [W1, W2, W3, W4]
    bs = [b0, b1, b2, b3]
    for i in range(4):
        aggs = _sc_segsum(*us, srcp, dstp)
        us = _tc_mid(aggs, us, dinv, bs[i].reshape(1, C), ws[i])
    aggs = _sc_segsum(*us, srcp, dstp)

    lg, ps = _tc_head(aggs, us, dinv, b4.reshape(1, C),
                      Wp1, bp1.reshape(1, 128), Wp2, bp2.reshape(1, 4))

    lp = lg[:N].reshape(N, B, 4).transpose(1, 0, 2).reshape(B, N * 4)

    policy, value = _tc_final(lp, ps, Wv1, bv1.reshape(1, 256),
                              Wv2, bv2.reshape(1, 64), Wv3, bv3.reshape(1, 1))
    return policy, value


# speedup vs baseline: 1.7275x; 1.0329x over previous
import functools

import jax
import jax.numpy as jnp
from jax import lax
from jax.experimental import pallas as pl
from jax.experimental.pallas import tpu as pltpu
from jax.experimental.pallas import tpu_sc as plsc

N = 10000
NP = 10240
B = 8
C = 32
E = 160000
EP = 163840
CHUNKS = EP // 16 // 128
BLKN = 1024
GRID = NP // BLKN
Q = 64


@functools.cache
def _mesh():
    return plsc.VectorSubcoreMesh(core_axis_name="c", subcore_axis_name="s")


def _sc_degree(dstp):
    def body(dst_ref, hist_ref, acc, zbuf, didx, ones, tb, isem, ssem):
        c = lax.axis_index("c")
        s = lax.axis_index("s")
        ebase = pl.multiple_of(s * (EP // 16), 128)
        hidx = pltpu.async_copy(dst_ref.at[pl.ds(ebase, EP // 16)], didx,
                                isem)
        for i in range(8):
            zbuf[pl.ds(i * 16, 16)] = jnp.zeros((16,), jnp.float32)
            ones[pl.ds(i * 16, 16)] = jnp.ones((16,), jnp.float32)
        for i in range(5):
            off = pl.multiple_of(s * 640 + i * 128, 128)
            pltpu.sync_copy(zbuf, acc.at[pl.ds(off, 128)])
        hidx.wait()
        plsc.subcore_barrier()

        hs = []
        for k in range(CHUNKS):
            off = pl.multiple_of(k * 128, 128)
            hs.append(pltpu.async_copy(ones, acc.at[didx.at[pl.ds(off, 128)]],
                                       ssem, add=True))
        for h in hs:
            h.wait()
        plsc.subcore_barrier()

        off = pl.multiple_of(c * (NP // 2) + s * 320, 64)
        pltpu.sync_copy(acc.at[pl.ds(off, 320)], tb)
        pltpu.sync_copy(tb, hist_ref.at[pl.ds(off, 320)])

    f = pl.kernel(
        body,
        out_type=jax.ShapeDtypeStruct((NP,), jnp.float32),
        mesh=_mesh(),
        scratch_types=[
            pltpu.VMEM_SHARED((NP,), jnp.float32),
            pltpu.VMEM((128,), jnp.float32),
            pltpu.VMEM((EP // 16,), jnp.int32),
            pltpu.VMEM((128,), jnp.float32),
            pltpu.VMEM((320,), jnp.float32),
            pltpu.SemaphoreType.DMA,
            pltpu.SemaphoreType.DMA,
        ],
    )
    return f(dstp)


_D = 3
_LAG = 2
_CW = 128
_NCH = EP // 16 // _CW


def _sc_segsum(u0, u1, u2, u3, srcp, dstp):
    def body(u0_ref, u1_ref, u2_ref, u3_ref, src_ref, dst_ref,
             a0_ref, a1_ref, a2_ref, a3_ref,
             usp, acc, buf, zbuf, sidx, didx, isem,
             g0, g1, g2, g3, g4, s0, s1, s2, s3, s4):
        c = lax.axis_index("c")
        s = lax.axis_index("s")
        ebase = pl.multiple_of(s * (EP // 16), 128)
        roff = pl.multiple_of(s * 640, 64)
        gsems = (g0, g1, g2, g3, g4)
        ssems = (s0, s1, s2, s3, s4)
        hsrc = pltpu.async_copy(src_ref.at[pl.ds(ebase, EP // 16)], sidx,
                                isem)
        hdst = pltpu.async_copy(dst_ref.at[pl.ds(ebase, EP // 16)], didx,
                                isem)

        for j in range(Q // 16):
            for i in range(32):
                zbuf[i, pl.ds(j * 16, 16)] = jnp.zeros((16,), jnp.float32)
        hsrc.wait()
        hdst.wait()

        def quarter(u_ref, out_ref):
            hst = pltpu.async_copy(u_ref.at[pl.ds(roff, 640)],
                                   usp.at[pl.ds(roff, 640)], isem)
            zh = []
            for i in range(20):
                off = pl.multiple_of(s * 640 + i * 32, 32)
                sem = gsems[i % _D] if i % 2 == 0 else ssems[(i // 2) % _D]
                zh.append(pltpu.async_copy(zbuf, acc.at[pl.ds(off, 32)], sem))
            hst.wait()
            for h in zh:
                h.wait()
            plsc.subcore_barrier()

            gh = [None] * _NCH
            sh = [None] * _NCH

            def gissue(k):
                slot = k % _D
                off = pl.multiple_of(k * _CW, 128)
                return pltpu.async_copy(
                    usp.at[sidx.at[pl.ds(off, _CW)]],
                    buf.at[pl.ds(slot * _CW, _CW)], gsems[slot])

            def sissue(k):
                slot = k % _D
                off = pl.multiple_of(k * _CW, 128)
                return pltpu.async_copy(
                    buf.at[pl.ds(slot * _CW, _CW)],
                    acc.at[didx.at[pl.ds(off, _CW)]], ssems[slot], add=True)

            for k in range(_NCH):
                if k >= _D:
                    sh[k - _D].wait()
                gh[k] = gissue(k)
                j = k - _LAG
                if j >= 0:
                    gh[j].wait()
                    sh[j] = sissue(j)
            for j in range(_NCH - _LAG, _NCH):
                gh[j].wait()
                sh[j] = sissue(j)
            for j in range(_NCH - _D, _NCH):
                sh[j].wait()
            plsc.subcore_barrier()

            pltpu.sync_copy(acc.at[pl.ds(roff, 640)],
                            out_ref.at[pl.ds(roff, 640)])
            plsc.subcore_barrier()

        @pl.when(c == 0)
        def _():
            quarter(u0_ref, a0_ref)
            quarter(u1_ref, a1_ref)

        @pl.when(c == 1)
        def _():
            quarter(u2_ref, a2_ref)
            quarter(u3_ref, a3_ref)

    f = pl.kernel(
        body,
        out_type=[jax.ShapeDtypeStruct((NP, Q), jnp.float32)] * 4,
        mesh=_mesh(),
        compiler_params=pltpu.CompilerParams(use_tc_tiling_on_sc=False),
        scratch_types=[
            pltpu.VMEM_SHARED((NP, Q), jnp.float32),
            pltpu.VMEM_SHARED((NP, Q), jnp.float32),
            pltpu.VMEM((_D * _CW, Q), jnp.float32),
            pltpu.VMEM((32, Q), jnp.float32),
            pltpu.VMEM((EP // 16,), jnp.int32),
            pltpu.VMEM((EP // 16,), jnp.int32),
            pltpu.SemaphoreType.DMA,
            pltpu.SemaphoreType.DMA,
            pltpu.SemaphoreType.DMA,
            pltpu.SemaphoreType.DMA,
            pltpu.SemaphoreType.DMA,
            pltpu.SemaphoreType.DMA,
            pltpu.SemaphoreType.DMA,
            pltpu.SemaphoreType.DMA,
            pltpu.SemaphoreType.DMA,
            pltpu.SemaphoreType.DMA,
            pltpu.SemaphoreType.DMA,
        ],
    )
    return f(u0, u1, u2, u3, srcp, dstp)


def _tc_l0(xp, hist, w0p):
    def body(xp_ref, hist_ref, w_ref, u0_ref, u1_ref, u2_ref, u3_ref,
             dinv_ref):
        pid = pl.program_id(0)
        rows = lax.broadcasted_iota(jnp.int32, (BLKN, 1), 0) + pid * BLKN
        dinv = jnp.where(rows < N, lax.rsqrt(hist_ref[...] + 1.0), 0.0)
        dinv_ref[...] = dinv
        xb = xp_ref[...]
        w = w_ref[...]
        outs = (u0_ref, u1_ref, u2_ref, u3_ref)
        for b in range(B):
            hw = jnp.dot(xb[:, b * 8:(b + 1) * 8], w,
                         preferred_element_type=jnp.float32)
            u = hw * dinv
            outs[b // 2][:, (b % 2) * C:(b % 2 + 1) * C] = u

    return pl.pallas_call(
        body,
        grid=(GRID,),
        in_specs=[
            pl.BlockSpec((BLKN, 64), lambda i: (i, 0)),
            pl.BlockSpec((BLKN, 1), lambda i: (i, 0)),
            pl.BlockSpec((8, C), lambda i: (0, 0)),
        ],
        out_specs=[pl.BlockSpec((BLKN, Q), lambda i: (i, 0))] * 4
        + [pl.BlockSpec((BLKN, 1), lambda i: (i, 0))],
        out_shape=[jax.ShapeDtypeStruct((NP, Q), jnp.float32)] * 4
        + [jax.ShapeDtypeStruct((NP, 1), jnp.float32)],
    )(xp, hist, w0p)


def _tc_mid(aggs, us, dinv, bprev, w):
    def body(a0_ref, a1_ref, a2_ref, a3_ref, u0_ref, u1_ref, u2_ref, u3_ref,
             dinv_ref, b_ref, w_ref, o0_ref, o1_ref, o2_ref, o3_ref):
        dv = dinv_ref[...]
        bb = b_ref[...]
        w = w_ref[...]
        a_refs = (a0_ref, a1_ref, a2_ref, a3_ref)
        u_refs = (u0_ref, u1_ref, u2_ref, u3_ref)
        o_refs = (o0_ref, o1_ref, o2_ref, o3_ref)
        for q in range(4):
            av = a_refs[q][...]
            uv = u_refs[q][...]
            for g in range(2):
                sl = slice(g * C, (g + 1) * C)
                h = jnp.maximum(dv * (av[:, sl] + uv[:, sl]) + bb, 0.0)
                o_refs[q][:, sl] = dv * jnp.dot(
                    h, w, preferred_element_type=jnp.float32)

    blk = pl.BlockSpec((BLKN, Q), lambda i: (i, 0))
    return pl.pallas_call(
        body,
        grid=(GRID,),
        in_specs=[blk] * 8 + [
            pl.BlockSpec((BLKN, 1), lambda i: (i, 0)),
            pl.BlockSpec((1, C), lambda i: (0, 0)),
            pl.BlockSpec((C, C), lambda i: (0, 0)),
        ],
        out_specs=[blk] * 4,
        out_shape=[jax.ShapeDtypeStruct((NP, Q), jnp.float32)] * 4,
    )(*aggs, *us, dinv, bprev, w)


def _tc_head(aggs, us, dinv, b4, wp1, bp1, wp2, bp2):
    def body(a0_ref, a1_ref, a2_ref, a3_ref, u0_ref, u1_ref, u2_ref, u3_ref,
             dinv_ref, b_ref, wp1_ref, bp1_ref, wp2_ref, bp2_ref,
             lg_ref, ps_ref):
        pid = pl.program_id(0)
        rows = lax.broadcasted_iota(jnp.int32, (BLKN, 1), 0) + pid * BLKN
        mask = rows < N
        dv = dinv_ref[...]
        bb = b_ref[...]
        wp1 = wp1_ref[...]
        wp2 = wp2_ref[...]
        a_refs = (a0_ref, a1_ref, a2_ref, a3_ref)
        u_refs = (u0_ref, u1_ref, u2_ref, u3_ref)
        sums = []
        for b in range(B):
            q, g = b // 2, b % 2
            sl = slice(g * C, (g + 1) * C)
            h = jnp.maximum(dv * (a_refs[q][:, sl] + u_refs[q][:, sl]) + bb,
                            0.0)
            z = jnp.maximum(jnp.dot(h, wp1, preferred_element_type=jnp.float32)
                            + bp1_ref[...], 0.0)
            lg = jnp.dot(z, wp2, preferred_element_type=jnp.float32) \
                + bp2_ref[...]
            lg_ref[:, b * 4:(b + 1) * 4] = lg
            hm = jnp.where(mask, h, 0.0)
            sums.append(jnp.sum(hm, axis=0, keepdims=True))
        part = jnp.concatenate(sums, axis=0)

        @pl.when(pid == 0)
        def _():
            ps_ref[...] = jnp.zeros_like(ps_ref)

        ps_ref[...] += part

    blk = pl.BlockSpec((BLKN, Q), lambda i: (i, 0))
    return pl.pallas_call(
        body,
        grid=(GRID,),
        in_specs=[blk] * 8 + [
            pl.BlockSpec((BLKN, 1), lambda i: (i, 0)),
            pl.BlockSpec((1, C), lambda i: (0, 0)),
            pl.BlockSpec((C, 128), lambda i: (0, 0)),
            pl.BlockSpec((1, 128), lambda i: (0, 0)),
            pl.BlockSpec((128, 4), lambda i: (0, 0)),
            pl.BlockSpec((1, 4), lambda i: (0, 0)),
        ],
        out_specs=[
            pl.BlockSpec((BLKN, 32), lambda i: (i, 0)),
            pl.BlockSpec((B, C), lambda i: (0, 0)),
        ],
        out_shape=[
            jax.ShapeDtypeStruct((NP, 32), jnp.float32),
            jax.ShapeDtypeStruct((B, C), jnp.float32),
        ],
    )(*aggs, *us, dinv, b4, wp1, bp1, wp2, bp2)


def _tc_final(lp, ps, wv1, bv1, wv2, bv2, wv3, bv3):
    def body(lp_ref, ps_ref, wv1_ref, bv1_ref, wv2_ref, bv2_ref,
             wv3_ref, bv3_ref, pol_ref, val_ref):
        lp = lp_ref[...]
        m = jnp.max(lp, axis=1, keepdims=True)
        e = jnp.exp(lp - m)
        pol_ref[...] = e / jnp.sum(e, axis=1, keepdims=True)
        pooled = ps_ref[...] * (1.0 / N)
        v = jnp.maximum(jnp.dot(pooled, wv1_ref[...],
                                preferred_element_type=jnp.float32)
                        + bv1_ref[...], 0.0)
        v = jnp.maximum(jnp.dot(v, wv2_ref[...],
                                preferred_element_type=jnp.float32)
                        + bv2_ref[...], 0.0)
        val_ref[...] = jnp.tanh(jnp.dot(v, wv3_ref[...],
                                        preferred_element_type=jnp.float32)
                                + bv3_ref[...])

    return pl.pallas_call(
        body,
        out_shape=[
            jax.ShapeDtypeStruct((B, N * 4), jnp.float32),
            jax.ShapeDtypeStruct((B, 1), jnp.float32),
        ],
    )(lp, ps, wv1, bv1, wv2, bv2, wv3, bv3)


def kernel(x, edge_index, W0, b0, W1, b1, W2, b2, W3, b3, W4, b4,
           Wp1, bp1, Wp2, bp2, Wv1, bv1, Wv2, bv2, Wv3, bv3):
    xt = jnp.transpose(x, (1, 0, 2))
    xt = jnp.pad(xt, ((0, NP - N), (0, 0), (0, 3)))
    xp = xt.reshape(NP, B * 8)
    w0p = jnp.pad(W0, ((0, 3), (0, 0)))

    src = edge_index[0]
    dst = edge_index[1]
    pad = jnp.full((EP - E,), NP - 1, dtype=src.dtype)
    srcp = jnp.concatenate([src, pad])
    dstp = jnp.concatenate([dst, pad])

    hist = _sc_degree(dstp).reshape(NP, 1)
    *us, dinv = _tc_l0(xp, hist, w0p)

    ws = [W1, W2, W3, W4]
    bs = [b0, b1, b2, b3]
    for i in range(4):
        aggs = _sc_segsum(*us, srcp, dstp)
        us = _tc_mid(aggs, us, dinv, bs[i].reshape(1, C), ws[i])
    aggs = _sc_segsum(*us, srcp, dstp)

    lg, ps = _tc_head(aggs, us, dinv, b4.reshape(1, C),
                      Wp1, bp1.reshape(1, 128), Wp2, bp2.reshape(1, 4))

    lp = lg[:N].reshape(N, B, 4).transpose(1, 0, 2).reshape(B, N * 4)

    policy, value = _tc_final(lp, ps, Wv1, bv1.reshape(1, 256),
                              Wv2, bv2.reshape(1, 64), Wv3, bv3.reshape(1, 1))
    return policy, value
